# baseline (speedup 1.0000x reference)
import functools

import jax
import jax.numpy as jnp
from jax import lax
from jax.experimental import pallas as pl
from jax.experimental.pallas import tpu as pltpu
from jax.experimental.pallas import tpu_sc as plsc

N = 10000
E = 320000
D = 128
NC = 2
NS = 16
NW = NC * NS
EPT = E // NW
CH = 80
NCHUNK = EPT // CH
RPT = 640
NPAD = NS * RPT
BN = 1000
NBUF = 2
STEP = 2 * NBUF
NITER = NCHUNK // STEP
NTAIL = NCHUNK - NITER * STEP

_mesh = plsc.VectorSubcoreMesh(
    core_axis_name="c", subcore_axis_name="s", num_cores=NC, num_subcores=NS
)


def _make_sc_pass(with_deg):
    def body(h_hbm, src_hbm, dst_hbm, zrows_hbm, *rest):
        if with_deg:
            ones_hbm, out_hbm, degp_hbm, acc_sh = rest[:4]
            scr = rest[4:]
        else:
            out_hbm, acc_sh = rest[:2]
            scr = rest[2:]
        c = lax.axis_index("c")
        s = lax.axis_index("s")
        wid = c * NS + s
        nrep = RPT // CH
        bufs = scr[0:2]
        src_ib = (scr[2:4], scr[4:6])
        dst_ib = (scr[6:8], scr[8:10])
        gsems = scr[10:12]
        ssems = scr[12:14]
        isems = scr[14:16]
        allsems = list(scr[10:16])
        b0 = bufs[0]

        def zero_acc():
            pltpu.sync_copy(zrows_hbm, b0)
            zd = [pltpu.make_async_copy(
                      b0, acc_sh.at[pl.ds(s * RPT + r * CH, CH)],
                      allsems[r % len(allsems)])
                  for r in range(nrep)]
            for d in zd:
                d.start()
            for d in zd:
                d.wait()

        def i_start(k, p, j, with_src):
            off = wid * EPT + k * CH
            if with_src:
                pltpu.async_copy(src_hbm.at[pl.ds(off, CH)], src_ib[p][j],
                                 isems[p])
            pltpu.async_copy(dst_hbm.at[pl.ds(off, CH)], dst_ib[p][j],
                             isems[p])

        def i_wait(k, p, j, with_src):
            off = wid * EPT + k * CH
            if with_src:
                pltpu.make_async_copy(src_hbm.at[pl.ds(off, CH)], src_ib[p][j],
                                      isems[p]).wait()
            pltpu.make_async_copy(dst_hbm.at[pl.ds(off, CH)], dst_ib[p][j],
                                  isems[p]).wait()

        def g_start(p, j):
            pltpu.async_copy(h_hbm.at[src_ib[p][j]], bufs[j], gsems[j])

        def g_wait(p, j):
            pltpu.make_async_copy(h_hbm.at[src_ib[p][j]], bufs[j],
                                  gsems[j]).wait()

        def s_start(p, j, buf=None):
            pltpu.async_copy(buf if buf is not None else bufs[j],
                             acc_sh.at[dst_ib[p][j]], ssems[j], add=True)

        def s_wait(p, j, buf=None):
            pltpu.make_async_copy(buf if buf is not None else bufs[j],
                                  acc_sh.at[dst_ib[p][j]], ssems[j]).wait()

        def dump(dst_hbm_out):
            rd = [pltpu.make_async_copy(
                      acc_sh.at[pl.ds(s * RPT + r * CH, CH)], bufs[r % 2],
                      gsems[r % 2])
                  for r in range(nrep)]
            wr = [pltpu.make_async_copy(
                      bufs[r % 2],
                      dst_hbm_out.at[pl.ds(wid * RPT + r * CH, CH)],
                      ssems[r % 2])
                  for r in range(nrep)]
            for r in range(nrep):
                if r >= 2:
                    wr[r - 2].wait()
                rd[r].start()
                rd[r].wait()
                wr[r].start()
            for r in range(nrep - 2, nrep):
                wr[r].wait()

        def scatter_phase(gather):
            cbuf = None if gather else b0
            for j in range(NBUF):
                i_start(j, 0, j, gather)
            for j in range(NBUF):
                i_wait(j, 0, j, gather)
                if gather:
                    g_start(0, j)
                else:
                    s_start(0, j, buf=cbuf)
            for j in range(NBUF):
                i_start(NBUF + j, 1, j, gather)

            def step(k4, carry):
                k = k4 * STEP
                for p in range(2):
                    base = k + p * NBUF
                    np_ = 1 - p
                    if gather:
                        for j in range(NBUF):
                            g_wait(p, j)
                            s_start(p, j)
                    for j in range(NBUF):
                        s_wait(p, j, buf=cbuf)
                        q = base + j + NBUF

                        @pl.when(q < NCHUNK)
                        def _(q=q, np_=np_, j=j):
                            i_wait(q, np_, j, gather)
                            if gather:
                                g_start(np_, j)
                            else:
                                s_start(np_, j, buf=cbuf)
                    for j in range(NBUF):
                        q2 = base + 2 * NBUF + j

                        @pl.when(q2 < NCHUNK)
                        def _(q2=q2, p=p, j=j):
                            i_start(q2, p, j, gather)
                return carry

            lax.fori_loop(0, NITER, step, 0)
            for t in range(NTAIL):
                q = NITER * STEP + t
                p = (q // NBUF) % 2
                j = q % NBUF
                if gather:
                    g_wait(p, j)
                    s_start(p, j)
            for t in range(NTAIL):
                q = NITER * STEP + t
                p = (q // NBUF) % 2
                j = q % NBUF
                s_wait(p, j, buf=cbuf)

        if with_deg:
            zero_acc()
            pltpu.sync_copy(ones_hbm, b0)
            plsc.subcore_barrier()
            scatter_phase(False)
            plsc.subcore_barrier()
            dump(degp_hbm)
            plsc.subcore_barrier()

        zero_acc()
        plsc.subcore_barrier()
        scatter_phase(True)
        plsc.subcore_barrier()
        dump(out_hbm)

    out_type = [jax.ShapeDtypeStruct((NC * NPAD, D), jnp.float32)]
    if with_deg:
        out_type.append(jax.ShapeDtypeStruct((NC * NPAD, D), jnp.float32))
    return pl.kernel(
        body,
        out_type=tuple(out_type) if with_deg else out_type[0],
        mesh=_mesh,
        scratch_types=(
            [pltpu.VMEM_SHARED((NPAD, D), jnp.float32)]
            + [pltpu.VMEM((CH, D), jnp.float32)] * 2
            + [pltpu.VMEM((CH,), jnp.int32)] * 4
            + [pltpu.VMEM((CH,), jnp.int32)] * 4
            + [pltpu.SemaphoreType.DMA] * 6
        ),
    )


_sc_agg_deg = _make_sc_pass(True)
_sc_agg = _make_sc_pass(False)


def _first_body(p0_ref, p1_ref, d0_ref, d1_ref, w_ref, b_ref, x_ref,
                o_ref, r_ref):
    deg = d0_ref[0][:, 0] + d1_ref[0][:, 0]
    recip = 1.0 / jnp.maximum(deg, 1.0)
    agg = (p0_ref[0] + p1_ref[0]) * recip[:, None]
    z = jnp.dot(agg, w_ref[...], preferred_element_type=jnp.float32) + b_ref[...]
    o_ref[...] = jnp.maximum(z, 0.0) + x_ref[...]
    r_ref[...] = recip[:, None]


def _mid_math(p0_ref, p1_ref, r_ref, w_ref, b_ref, x_ref):
    agg = (p0_ref[0] + p1_ref[0]) * r_ref[...]
    z = jnp.dot(agg, w_ref[...], preferred_element_type=jnp.float32) + b_ref[...]
    return jnp.maximum(z, 0.0) + x_ref[...]


def _mid_body(p0_ref, p1_ref, r_ref, w_ref, b_ref, x_ref, o_ref):
    o_ref[...] = _mid_math(p0_ref, p1_ref, r_ref, w_ref, b_ref, x_ref)


def _final_body(p0_ref, p1_ref, r_ref, w_ref, b_ref, x_ref,
                wfc_ref, bfc_ref, o_ref):
    h = _mid_math(p0_ref, p1_ref, r_ref, w_ref, b_ref, x_ref)
    o_ref[...] = jnp.sum(h * wfc_ref[...], axis=1, keepdims=True) + bfc_ref[...]


_p_specs = [
    pl.BlockSpec((1, BN, D), lambda i: (0, i, 0)),
    pl.BlockSpec((1, BN, D), lambda i: (1, i, 0)),
]
_wbx_specs = [
    pl.BlockSpec((D, D), lambda i: (0, 0)),
    pl.BlockSpec((1, D), lambda i: (0, 0)),
    pl.BlockSpec((BN, D), lambda i: (i, 0)),
]
_r_spec = pl.BlockSpec((BN, 1), lambda i: (i, 0))

_first = pl.pallas_call(
    _first_body,
    grid=(N // BN,),
    in_specs=_p_specs + _p_specs + _wbx_specs,
    out_specs=[pl.BlockSpec((BN, D), lambda i: (i, 0)), _r_spec],
    out_shape=[jax.ShapeDtypeStruct((N, D), jnp.float32),
               jax.ShapeDtypeStruct((N, 1), jnp.float32)],
)

_mid = pl.pallas_call(
    _mid_body,
    grid=(N // BN,),
    in_specs=_p_specs + [_r_spec] + _wbx_specs,
    out_specs=pl.BlockSpec((BN, D), lambda i: (i, 0)),
    out_shape=jax.ShapeDtypeStruct((N, D), jnp.float32),
)

_final = pl.pallas_call(
    _final_body,
    grid=(N // BN,),
    in_specs=_p_specs + [_r_spec] + _wbx_specs + [
        pl.BlockSpec((1, D), lambda i: (0, 0)),
        pl.BlockSpec((1, 1), lambda i: (0, 0)),
    ],
    out_specs=pl.BlockSpec((BN, 1), lambda i: (i, 0)),
    out_shape=jax.ShapeDtypeStruct((N, 1), jnp.float32),
)


def kernel(x, edge_index, W0, b0, W1, b1, W2, b2, W3, b3, Wfc, bfc):
    src = edge_index[0]
    dst = edge_index[1]
    zrows = jnp.zeros((CH, D), jnp.float32)
    ones = jnp.ones((CH, D), jnp.float32)

    P, degp = _sc_agg_deg(x, src, dst, zrows, ones)
    P = P.reshape(NC, NPAD, D)
    degp = degp.reshape(NC, NPAD, D)
    h, recip = _first(P, P, degp, degp, W0, b0.reshape(1, D), x)
    for W, b in ((W1, b1), (W2, b2)):
        P = _sc_agg(h, src, dst, zrows).reshape(NC, NPAD, D)
        h = _mid(P, P, recip, W, b.reshape(1, D), x)
    P = _sc_agg(h, src, dst, zrows).reshape(NC, NPAD, D)
    out = _final(P, P, recip, W3, b3.reshape(1, D), x,
                 Wfc.reshape(1, D), bfc.reshape(1, 1))
    return out[:, 0]


# speedup vs baseline: 1.1582x; 1.1582x over previous
import functools

import jax
import jax.numpy as jnp
from jax import lax
from jax.experimental import pallas as pl
from jax.experimental.pallas import tpu as pltpu
from jax.experimental.pallas import tpu_sc as plsc

N = 10000
E = 320000
D = 128
NC = 2
NS = 16
NW = NC * NS
EPT = E // NW
CH = 80
NCHUNK = EPT // CH
RPT = 640
NPAD = NS * RPT
BN = 1000
NBUF = 3
NITER = NCHUNK // NBUF
NTAIL = NCHUNK - NITER * NBUF

_mesh = plsc.VectorSubcoreMesh(
    core_axis_name="c", subcore_axis_name="s", num_cores=NC, num_subcores=NS
)


def _make_sc_pass(with_deg):
    def body(h_hbm, src_hbm, dst_hbm, zrows_hbm, *rest):
        if with_deg:
            ones_hbm, out_hbm, degp_hbm, acc_sh = rest[:4]
            scr = rest[4:]
        else:
            out_hbm, acc_sh = rest[:2]
            scr = rest[2:]
        c = lax.axis_index("c")
        s = lax.axis_index("s")
        wid = c * NS + s
        nrep = RPT // CH
        bufs = scr[:NBUF]
        src_b = scr[NBUF:2 * NBUF]
        dst_b = scr[2 * NBUF:3 * NBUF]
        gsems = scr[3 * NBUF:4 * NBUF]
        ssems = scr[4 * NBUF:5 * NBUF]
        isems = scr[5 * NBUF:6 * NBUF]
        b0 = bufs[0]

        def zero_acc():
            pltpu.sync_copy(zrows_hbm, b0)
            zd = [pltpu.make_async_copy(
                      b0, acc_sh.at[pl.ds(s * RPT + r * CH, CH)],
                      scr[3 * NBUF + r % (3 * NBUF)])
                  for r in range(nrep)]
            for d in zd:
                d.start()
            for d in zd:
                d.wait()

        def i_start(k, j, with_src):
            off = wid * EPT + k * CH
            if with_src:
                pltpu.async_copy(src_hbm.at[pl.ds(off, CH)], src_b[j], isems[j])
            pltpu.async_copy(dst_hbm.at[pl.ds(off, CH)], dst_b[j], isems[j])

        def i_wait(k, j, with_src):
            off = wid * EPT + k * CH
            if with_src:
                pltpu.make_async_copy(src_hbm.at[pl.ds(off, CH)], src_b[j],
                                      isems[j]).wait()
            pltpu.make_async_copy(dst_hbm.at[pl.ds(off, CH)], dst_b[j],
                                  isems[j]).wait()

        def g_start(j):
            pltpu.async_copy(h_hbm.at[src_b[j]], bufs[j], gsems[j])

        def g_wait(j):
            pltpu.make_async_copy(h_hbm.at[src_b[j]], bufs[j], gsems[j]).wait()

        def s_start(j, buf=None):
            pltpu.async_copy(buf if buf is not None else bufs[j],
                             acc_sh.at[dst_b[j]], ssems[j], add=True)

        def s_wait(j, buf=None):
            pltpu.make_async_copy(buf if buf is not None else bufs[j],
                                  acc_sh.at[dst_b[j]], ssems[j]).wait()

        def dump(dst_hbm_out):
            rd = [pltpu.make_async_copy(
                      acc_sh.at[pl.ds(s * RPT + r * CH, CH)], bufs[r % 2],
                      gsems[r % 2])
                  for r in range(nrep)]
            wr = [pltpu.make_async_copy(
                      bufs[r % 2],
                      dst_hbm_out.at[pl.ds(wid * RPT + r * CH, CH)],
                      ssems[r % 2])
                  for r in range(nrep)]
            for r in range(nrep):
                if r >= 2:
                    wr[r - 2].wait()
                rd[r].start()
                rd[r].wait()
                wr[r].start()
            for r in range(nrep - 2, nrep):
                wr[r].wait()

        def scatter_phase(gather):
            for j in range(NBUF):
                i_start(j, j, gather)
            for j in range(NBUF):
                i_wait(j, j, gather)
                if gather:
                    g_start(j)
                else:
                    s_start(j, buf=b0)

            if gather:
                def step(k4, carry):
                    k = k4 * NBUF
                    for j in range(NBUF):
                        g_wait(j)
                        s_start(j)
                    for j in range(NBUF):
                        s_wait(j)

                        @pl.when(k + j + NBUF < NCHUNK)
                        def _(j=j, k=k):
                            i_start(k + j + NBUF, j, True)
                            i_wait(k + j + NBUF, j, True)
                            g_start(j)
                    return carry

                lax.fori_loop(0, NITER, step, 0)
                for t in range(NTAIL):
                    g_wait(t)
                    s_start(t)
                for t in range(NTAIL):
                    s_wait(t)
            else:
                def step(k4, carry):
                    k = k4 * NBUF
                    for j in range(NBUF):
                        s_wait(j, buf=b0)

                        @pl.when(k + j + NBUF < NCHUNK)
                        def _(j=j, k=k):
                            i_start(k + j + NBUF, j, False)
                            i_wait(k + j + NBUF, j, False)
                            s_start(j, buf=b0)
                    return carry

                lax.fori_loop(0, NITER, step, 0)
                for t in range(NTAIL):
                    s_wait(t, buf=b0)

        if with_deg:
            zero_acc()
            pltpu.sync_copy(ones_hbm, b0)
            plsc.subcore_barrier()
            scatter_phase(False)
            plsc.subcore_barrier()
            dump(degp_hbm)
            plsc.subcore_barrier()

        zero_acc()
        plsc.subcore_barrier()
        scatter_phase(True)
        plsc.subcore_barrier()
        dump(out_hbm)

    out_type = [jax.ShapeDtypeStruct((NC * NPAD, D), jnp.float32)]
    if with_deg:
        out_type.append(jax.ShapeDtypeStruct((NC * NPAD, D), jnp.float32))
    return pl.kernel(
        body,
        out_type=tuple(out_type) if with_deg else out_type[0],
        mesh=_mesh,
        scratch_types=(
            [pltpu.VMEM_SHARED((NPAD, D), jnp.float32)]
            + [pltpu.VMEM((CH, D), jnp.float32)] * NBUF
            + [pltpu.VMEM((CH,), jnp.int32)] * NBUF
            + [pltpu.VMEM((CH,), jnp.int32)] * NBUF
            + [pltpu.SemaphoreType.DMA] * (3 * NBUF)
        ),
    )


_sc_agg_deg = _make_sc_pass(True)
_sc_agg = _make_sc_pass(False)


def _first_body(p0_ref, p1_ref, d0_ref, d1_ref, w_ref, b_ref, x_ref,
                o_ref, r_ref):
    deg = d0_ref[0][:, 0] + d1_ref[0][:, 0]
    recip = 1.0 / jnp.maximum(deg, 1.0)
    agg = (p0_ref[0] + p1_ref[0]) * recip[:, None]
    z = jnp.dot(agg, w_ref[...], preferred_element_type=jnp.float32) + b_ref[...]
    o_ref[...] = jnp.maximum(z, 0.0) + x_ref[...]
    r_ref[...] = recip[:, None]


def _mid_math(p0_ref, p1_ref, r_ref, w_ref, b_ref, x_ref):
    agg = (p0_ref[0] + p1_ref[0]) * r_ref[...]
    z = jnp.dot(agg, w_ref[...], preferred_element_type=jnp.float32) + b_ref[...]
    return jnp.maximum(z, 0.0) + x_ref[...]


def _mid_body(p0_ref, p1_ref, r_ref, w_ref, b_ref, x_ref, o_ref):
    o_ref[...] = _mid_math(p0_ref, p1_ref, r_ref, w_ref, b_ref, x_ref)


def _final_body(p0_ref, p1_ref, r_ref, w_ref, b_ref, x_ref,
                wfc_ref, bfc_ref, o_ref):
    h = _mid_math(p0_ref, p1_ref, r_ref, w_ref, b_ref, x_ref)
    o_ref[...] = jnp.sum(h * wfc_ref[...], axis=1, keepdims=True) + bfc_ref[...]


_p_specs = [
    pl.BlockSpec((1, BN, D), lambda i: (0, i, 0)),
    pl.BlockSpec((1, BN, D), lambda i: (1, i, 0)),
]
_wbx_specs = [
    pl.BlockSpec((D, D), lambda i: (0, 0)),
    pl.BlockSpec((1, D), lambda i: (0, 0)),
    pl.BlockSpec((BN, D), lambda i: (i, 0)),
]
_r_spec = pl.BlockSpec((BN, 1), lambda i: (i, 0))

_first = pl.pallas_call(
    _first_body,
    grid=(N // BN,),
    in_specs=_p_specs + _p_specs + _wbx_specs,
    out_specs=[pl.BlockSpec((BN, D), lambda i: (i, 0)), _r_spec],
    out_shape=[jax.ShapeDtypeStruct((N, D), jnp.float32),
               jax.ShapeDtypeStruct((N, 1), jnp.float32)],
)

_mid = pl.pallas_call(
    _mid_body,
    grid=(N // BN,),
    in_specs=_p_specs + [_r_spec] + _wbx_specs,
    out_specs=pl.BlockSpec((BN, D), lambda i: (i, 0)),
    out_shape=jax.ShapeDtypeStruct((N, D), jnp.float32),
)

_final = pl.pallas_call(
    _final_body,
    grid=(N // BN,),
    in_specs=_p_specs + [_r_spec] + _wbx_specs + [
        pl.BlockSpec((1, D), lambda i: (0, 0)),
        pl.BlockSpec((1, 1), lambda i: (0, 0)),
    ],
    out_specs=pl.BlockSpec((BN, 1), lambda i: (i, 0)),
    out_shape=jax.ShapeDtypeStruct((N, 1), jnp.float32),
)


def kernel(x, edge_index, W0, b0, W1, b1, W2, b2, W3, b3, Wfc, bfc):
    src = edge_index[0]
    dst = edge_index[1]
    zrows = jnp.zeros((CH, D), jnp.float32)
    ones = jnp.ones((CH, D), jnp.float32)

    P, degp = _sc_agg_deg(x, src, dst, zrows, ones)
    P = P.reshape(NC, NPAD, D)
    degp = degp.reshape(NC, NPAD, D)
    h, recip = _first(P, P, degp, degp, W0, b0.reshape(1, D), x)
    for W, b in ((W1, b1), (W2, b2)):
        P = _sc_agg(h, src, dst, zrows).reshape(NC, NPAD, D)
        h = _mid(P, P, recip, W, b.reshape(1, D), x)
    P = _sc_agg(h, src, dst, zrows).reshape(NC, NPAD, D)
    out = _final(P, P, recip, W3, b3.reshape(1, D), x,
                 Wfc.reshape(1, D), bfc.reshape(1, 1))
    return out[:, 0]
